# baseline (device time: 123019 ns/iter reference)
import functools

import jax
import jax.numpy as jnp
from jax import lax
from jax.experimental import pallas as pl
from jax.experimental.pallas import tpu as pltpu

N_DEV = 4
B_PER = 2
SQ = 512
SKV = 512
HQ = 32
H_PER = HQ // N_DEV
DH = 64
DM = 768
F_PER = H_PER * DH
WINDOW = 128
SCALE = 0.125
WQ_HALF = DM // 2
WO_HALF = F_PER // 2


def kernel(x, Wq, K_ext, V_ext, Wo):
    k_r = K_ext.reshape(N_DEV * B_PER, SKV, HQ * DH)
    v_r = V_ext.reshape(N_DEV * B_PER, SKV, HQ * DH)
    x_bf = x.astype(jnp.bfloat16)
    wq_bf = Wq.astype(jnp.bfloat16)
    wo_bf = Wo.astype(jnp.bfloat16)

    def body(x_ref, wq_ref, k_hbm, v_hbm, wo_ref, out_ref,
             wq_bufL, wq_bufR, wq_bufO, wo_bufL, wo_bufR, wo_bufO,
             k_scr, v_scr,
             p1_ssem, p1_rsem, p2_ssem, p2_rsem, kdma_sem, vdma_sem):
        my_pos = lax.axis_index("i")
        left = (my_pos - 1) % N_DEV
        right = (my_pos + 1) % N_DEV
        opp = (my_pos + 2) % N_DEV

        chunk_ids = [my_pos, left, right, opp]
        kv_dmas = []
        for i, cc in enumerate(chunk_ids):
            ck = pltpu.make_async_copy(
                k_hbm.at[pl.ds(my_pos * B_PER, B_PER), :,
                         pl.ds(cc * F_PER, F_PER)],
                k_scr.at[i], kdma_sem.at[i])
            cv = pltpu.make_async_copy(
                v_hbm.at[pl.ds(my_pos * B_PER, B_PER), :,
                         pl.ds(cc * F_PER, F_PER)],
                v_scr.at[i], vdma_sem.at[i])
            ck.start()
            cv.start()
            kv_dmas.append((ck, cv))

        barrier = pltpu.get_barrier_semaphore()
        for nbr in (left, right):
            pl.semaphore_signal(barrier, inc=1, device_id=(nbr,),
                                device_id_type=pl.DeviceIdType.MESH)
        pl.semaphore_wait(barrier, 2)

        qi = lax.broadcasted_iota(jnp.int32, (SQ, SKV), 0)
        ki = lax.broadcasted_iota(jnp.int32, (SQ, SKV), 1)
        maskf = (jnp.abs(qi - ki) <= WINDOW).astype(jnp.float32)

        def compute_chunk(i, wq_c, wo_c, first):
            ck, cv = kv_dmas[i]
            ck.wait()
            cv.wait()
            for b in range(B_PER):
                q = jnp.dot(x_ref[b], wq_c,
                            preferred_element_type=jnp.float32
                            ).astype(jnp.bfloat16)
                kb = k_scr[i, b].astype(jnp.bfloat16)
                vb = v_scr[i, b].astype(jnp.bfloat16)
                ctxs = []
                for hh in range(H_PER):
                    qh = q[:, hh * DH:(hh + 1) * DH]
                    s = lax.dot_general(
                        qh, kb[:, hh * DH:(hh + 1) * DH],
                        (((1,), (1,)), ((), ())),
                        preferred_element_type=jnp.float32)
                    w = jnp.exp(s * SCALE) * maskf
                    r = 1.0 / jnp.sum(w, axis=-1, keepdims=True)
                    ctx_h = jnp.dot(w.astype(jnp.bfloat16),
                                    vb[:, hh * DH:(hh + 1) * DH],
                                    preferred_element_type=jnp.float32)
                    ctxs.append((ctx_h * r).astype(jnp.bfloat16))
                ctx = jnp.concatenate(ctxs, axis=1)
                contrib = jnp.dot(ctx, wo_c, preferred_element_type=jnp.float32)
                if first:
                    out_ref[b] = contrib
                else:
                    out_ref[b] = out_ref[b] + contrib

        p1 = []
        for idx, (src, dst, tgt) in enumerate([
            (wq_ref, wq_bufL, right),
            (wq_ref, wq_bufR, left),
            (wo_ref, wo_bufL, right),
            (wo_ref, wo_bufR, left),
        ]):
            r = pltpu.make_async_remote_copy(
                src_ref=src, dst_ref=dst,
                send_sem=p1_ssem.at[idx], recv_sem=p1_rsem.at[idx],
                device_id=(tgt,), device_id_type=pl.DeviceIdType.MESH)
            r.start()
            p1.append(r)

        compute_chunk(0, wq_ref[...], wo_ref[...], first=True)

        for r in p1:
            r.wait_recv()

        p2 = []
        for idx, (src, dst, tgt) in enumerate([
            (wq_bufR.at[pl.ds(0, WQ_HALF)], wq_bufO.at[pl.ds(0, WQ_HALF)], left),
            (wq_bufL.at[pl.ds(WQ_HALF, WQ_HALF)],
             wq_bufO.at[pl.ds(WQ_HALF, WQ_HALF)], right),
            (wo_bufR.at[pl.ds(0, WO_HALF)], wo_bufO.at[pl.ds(0, WO_HALF)], left),
            (wo_bufL.at[pl.ds(WO_HALF, WO_HALF)],
             wo_bufO.at[pl.ds(WO_HALF, WO_HALF)], right),
        ]):
            r = pltpu.make_async_remote_copy(
                src_ref=src, dst_ref=dst,
                send_sem=p2_ssem.at[idx], recv_sem=p2_rsem.at[idx],
                device_id=(tgt,), device_id_type=pl.DeviceIdType.MESH)
            r.start()
            p2.append(r)

        compute_chunk(1, wq_bufL[...], wo_bufL[...], first=False)
        compute_chunk(2, wq_bufR[...], wo_bufR[...], first=False)

        for r in p2:
            r.wait_recv()

        compute_chunk(3, wq_bufO[...], wo_bufO[...], first=False)

        for r in p1 + p2:
            r.wait_send()

        @functools.partial(pl.run_scoped, sem=pltpu.SemaphoreType.REGULAR)
        def _(sem):
            for nbr in (left, right):
                pl.semaphore_signal(sem, inc=1, device_id=(nbr,),
                                    device_id_type=pl.DeviceIdType.MESH)
            pl.semaphore_wait(sem, 2)

    return pl.pallas_call(
        body,
        out_shape=jax.ShapeDtypeStruct((B_PER, SQ, DM), jnp.float32),
        in_specs=[
            pl.BlockSpec(memory_space=pltpu.VMEM),
            pl.BlockSpec(memory_space=pltpu.VMEM),
            pl.BlockSpec(memory_space=pltpu.MemorySpace.HBM),
            pl.BlockSpec(memory_space=pltpu.MemorySpace.HBM),
            pl.BlockSpec(memory_space=pltpu.VMEM),
        ],
        out_specs=pl.BlockSpec(memory_space=pltpu.VMEM),
        scratch_shapes=[
            pltpu.VMEM((DM, F_PER), jnp.bfloat16),
            pltpu.VMEM((DM, F_PER), jnp.bfloat16),
            pltpu.VMEM((DM, F_PER), jnp.bfloat16),
            pltpu.VMEM((F_PER, DM), jnp.bfloat16),
            pltpu.VMEM((F_PER, DM), jnp.bfloat16),
            pltpu.VMEM((F_PER, DM), jnp.bfloat16),
            pltpu.VMEM((N_DEV, B_PER, SKV, F_PER), jnp.float32),
            pltpu.VMEM((N_DEV, B_PER, SKV, F_PER), jnp.float32),
            pltpu.SemaphoreType.DMA((4,)),
            pltpu.SemaphoreType.DMA((4,)),
            pltpu.SemaphoreType.DMA((4,)),
            pltpu.SemaphoreType.DMA((4,)),
            pltpu.SemaphoreType.DMA((4,)),
            pltpu.SemaphoreType.DMA((4,)),
        ],
        compiler_params=pltpu.CompilerParams(
            collective_id=0, vmem_limit_bytes=100 * 1024 * 1024),
    )(x_bf, wq_bf, k_r, v_r, wo_bf)


# device time: 77366 ns/iter; 1.5901x vs baseline; 1.5901x over previous
import functools

import jax
import jax.numpy as jnp
from jax import lax
from jax.experimental import pallas as pl
from jax.experimental.pallas import tpu as pltpu

N_DEV = 4
B_PER = 2
SQ = 512
SKV = 512
HQ = 32
H_PER = HQ // N_DEV
DH = 64
DM = 768
F_PER = H_PER * DH
WINDOW = 128
SCALE = 0.125
WQ_HALF = DM // 2
WO_HALF = F_PER // 2


def kernel(x, Wq, K_ext, V_ext, Wo):
    my = lax.axis_index("i")

    k_s = lax.dynamic_slice_in_dim(K_ext, my * B_PER, B_PER, axis=0)
    v_s = lax.dynamic_slice_in_dim(V_ext, my * B_PER, B_PER, axis=0)
    k_bf = (k_s.transpose(2, 0, 1, 3)
            .reshape(N_DEV, H_PER, B_PER, SKV, DH).astype(jnp.bfloat16))
    v_bf = (v_s.transpose(2, 0, 1, 3)
            .reshape(N_DEV, H_PER, B_PER, SKV, DH).astype(jnp.bfloat16))
    x_bf = x.astype(jnp.bfloat16)
    wq_bf = (Wq * SCALE).astype(jnp.bfloat16)
    wo_bf = Wo.astype(jnp.bfloat16)

    def body(x_ref, wq_ref, k_ref, v_ref, wo_ref, out_ref,
             wq_bufL, wq_bufR, wq_bufO, wo_bufL, wo_bufR, wo_bufO,
             p1_ssem, p1_rsem, p2_ssem, p2_rsem):
        my_pos = lax.axis_index("i")
        left = (my_pos - 1) % N_DEV
        right = (my_pos + 1) % N_DEV
        opp = (my_pos + 2) % N_DEV

        barrier = pltpu.get_barrier_semaphore()
        for nbr in (left, right):
            pl.semaphore_signal(barrier, inc=1, device_id=(nbr,),
                                device_id_type=pl.DeviceIdType.MESH)
        pl.semaphore_wait(barrier, 2)

        qi = lax.broadcasted_iota(jnp.int32, (SQ, SKV), 0)
        ki = lax.broadcasted_iota(jnp.int32, (SQ, SKV), 1)
        maskf = (jnp.abs(qi - ki) <= WINDOW).astype(jnp.float32)

        def compute_chunk(c, wq_c, wo_c, first):
            for b in range(B_PER):
                q = jnp.dot(x_ref[b], wq_c,
                            preferred_element_type=jnp.float32
                            ).astype(jnp.bfloat16)
                ctxs = []
                for hh in range(H_PER):
                    qh = q[:, hh * DH:(hh + 1) * DH]
                    s = lax.dot_general(
                        qh, k_ref[c, hh, b], (((1,), (1,)), ((), ())),
                        preferred_element_type=jnp.float32)
                    w = jnp.exp(s) * maskf
                    r = 1.0 / jnp.sum(w, axis=-1, keepdims=True)
                    ctx_h = jnp.dot(w.astype(jnp.bfloat16), v_ref[c, hh, b],
                                    preferred_element_type=jnp.float32)
                    ctxs.append((ctx_h * r).astype(jnp.bfloat16))
                ctx = jnp.concatenate(ctxs, axis=1)
                contrib = jnp.dot(ctx, wo_c, preferred_element_type=jnp.float32)
                if first:
                    out_ref[b] = contrib
                else:
                    out_ref[b] = out_ref[b] + contrib

        p1 = []
        for idx, (src, dst, tgt) in enumerate([
            (wq_ref, wq_bufL, right),
            (wq_ref, wq_bufR, left),
            (wo_ref, wo_bufL, right),
            (wo_ref, wo_bufR, left),
        ]):
            r = pltpu.make_async_remote_copy(
                src_ref=src, dst_ref=dst,
                send_sem=p1_ssem.at[idx], recv_sem=p1_rsem.at[idx],
                device_id=(tgt,), device_id_type=pl.DeviceIdType.MESH)
            r.start()
            p1.append(r)

        compute_chunk(my_pos, wq_ref[...], wo_ref[...], first=True)

        for r in p1:
            r.wait_recv()

        p2 = []
        for idx, (src, dst, tgt) in enumerate([
            (wq_bufR.at[pl.ds(0, WQ_HALF)], wq_bufO.at[pl.ds(0, WQ_HALF)], left),
            (wq_bufL.at[pl.ds(WQ_HALF, WQ_HALF)],
             wq_bufO.at[pl.ds(WQ_HALF, WQ_HALF)], right),
            (wo_bufR.at[pl.ds(0, WO_HALF)], wo_bufO.at[pl.ds(0, WO_HALF)], left),
            (wo_bufL.at[pl.ds(WO_HALF, WO_HALF)],
             wo_bufO.at[pl.ds(WO_HALF, WO_HALF)], right),
        ]):
            r = pltpu.make_async_remote_copy(
                src_ref=src, dst_ref=dst,
                send_sem=p2_ssem.at[idx], recv_sem=p2_rsem.at[idx],
                device_id=(tgt,), device_id_type=pl.DeviceIdType.MESH)
            r.start()
            p2.append(r)

        compute_chunk(left, wq_bufL[...], wo_bufL[...], first=False)
        compute_chunk(right, wq_bufR[...], wo_bufR[...], first=False)

        for r in p2:
            r.wait_recv()

        compute_chunk(opp, wq_bufO[...], wo_bufO[...], first=False)

        for r in p1 + p2:
            r.wait_send()

        @functools.partial(pl.run_scoped, sem=pltpu.SemaphoreType.REGULAR)
        def _(sem):
            for nbr in (left, right):
                pl.semaphore_signal(sem, inc=1, device_id=(nbr,),
                                    device_id_type=pl.DeviceIdType.MESH)
            pl.semaphore_wait(sem, 2)

    return pl.pallas_call(
        body,
        out_shape=jax.ShapeDtypeStruct((B_PER, SQ, DM), jnp.float32),
        in_specs=[pl.BlockSpec(memory_space=pltpu.VMEM)] * 5,
        out_specs=pl.BlockSpec(memory_space=pltpu.VMEM),
        scratch_shapes=[
            pltpu.VMEM((DM, F_PER), jnp.bfloat16),
            pltpu.VMEM((DM, F_PER), jnp.bfloat16),
            pltpu.VMEM((DM, F_PER), jnp.bfloat16),
            pltpu.VMEM((F_PER, DM), jnp.bfloat16),
            pltpu.VMEM((F_PER, DM), jnp.bfloat16),
            pltpu.VMEM((F_PER, DM), jnp.bfloat16),
            pltpu.SemaphoreType.DMA((4,)),
            pltpu.SemaphoreType.DMA((4,)),
            pltpu.SemaphoreType.DMA((4,)),
            pltpu.SemaphoreType.DMA((4,)),
        ],
        compiler_params=pltpu.CompilerParams(
            collective_id=0, vmem_limit_bytes=100 * 1024 * 1024),
    )(x_bf, wq_bf, k_bf, v_bf, wo_bf)


# device time: 70157 ns/iter; 1.7535x vs baseline; 1.1028x over previous
import functools

import jax
import jax.numpy as jnp
from jax import lax
from jax.experimental import pallas as pl
from jax.experimental.pallas import tpu as pltpu

N_DEV = 4
B_PER = 2
SQ = 512
SKV = 512
HQ = 32
H_PER = HQ // N_DEV
DH = 64
DM = 768
F_PER = H_PER * DH
WINDOW = 128
SCALE = 0.125
WQ_HALF = DM // 2
WO_HALF = F_PER // 2


def kernel(x, Wq, K_ext, V_ext, Wo):
    my = lax.axis_index("i")

    k_bf = (lax.dynamic_slice_in_dim(K_ext, my * B_PER, B_PER, axis=0)
            .reshape(B_PER, SKV, HQ * DH).astype(jnp.bfloat16))
    v_bf = (lax.dynamic_slice_in_dim(V_ext, my * B_PER, B_PER, axis=0)
            .reshape(B_PER, SKV, HQ * DH).astype(jnp.bfloat16))
    x_bf = x.astype(jnp.bfloat16)
    wq_bf = (Wq * SCALE).astype(jnp.bfloat16)
    wo_bf = Wo.astype(jnp.bfloat16)

    def body(x_ref, wq_ref, k_hbm, v_hbm, wo_ref, out_ref,
             wq_bufL, wq_bufR, wq_bufO, wo_bufL, wo_bufR, wo_bufO,
             k_scr, v_scr,
             p1_ssem, p1_rsem, p2_ssem, p2_rsem, kdma_sem, vdma_sem):
        my_pos = lax.axis_index("i")
        left = (my_pos - 1) % N_DEV
        right = (my_pos + 1) % N_DEV
        opp = (my_pos + 2) % N_DEV

        kv_dmas = []
        for i, cc in enumerate([my_pos, left, right, opp]):
            ck = pltpu.make_async_copy(
                k_hbm.at[:, :, pl.ds(cc * F_PER, F_PER)],
                k_scr.at[i], kdma_sem.at[i])
            cv = pltpu.make_async_copy(
                v_hbm.at[:, :, pl.ds(cc * F_PER, F_PER)],
                v_scr.at[i], vdma_sem.at[i])
            ck.start()
            cv.start()
            kv_dmas.append((ck, cv))

        barrier = pltpu.get_barrier_semaphore()
        for nbr in (left, right):
            pl.semaphore_signal(barrier, inc=1, device_id=(nbr,),
                                device_id_type=pl.DeviceIdType.MESH)
        pl.semaphore_wait(barrier, 2)

        qi = lax.broadcasted_iota(jnp.int32, (SQ, SKV), 0)
        ki = lax.broadcasted_iota(jnp.int32, (SQ, SKV), 1)
        maskf = (jnp.abs(qi - ki) <= WINDOW).astype(jnp.float32)

        def compute_chunk(i, wq_c, wo_c, first):
            ck, cv = kv_dmas[i]
            ck.wait()
            cv.wait()
            for b in range(B_PER):
                q = jnp.dot(x_ref[b], wq_c,
                            preferred_element_type=jnp.float32
                            ).astype(jnp.bfloat16)
                ctxs = []
                for hh in range(H_PER):
                    qh = q[:, hh * DH:(hh + 1) * DH]
                    s = lax.dot_general(
                        qh, k_scr[i, b, :, hh * DH:(hh + 1) * DH],
                        (((1,), (1,)), ((), ())),
                        preferred_element_type=jnp.float32)
                    w = jnp.exp(s) * maskf
                    r = 1.0 / jnp.sum(w, axis=-1, keepdims=True)
                    ctx_h = jnp.dot(w.astype(jnp.bfloat16),
                                    v_scr[i, b, :, hh * DH:(hh + 1) * DH],
                                    preferred_element_type=jnp.float32)
                    ctxs.append((ctx_h * r).astype(jnp.bfloat16))
                ctx = jnp.concatenate(ctxs, axis=1)
                contrib = jnp.dot(ctx, wo_c, preferred_element_type=jnp.float32)
                if first:
                    out_ref[b] = contrib
                else:
                    out_ref[b] = out_ref[b] + contrib

        p1 = []
        for idx, (src, dst, tgt) in enumerate([
            (wq_ref, wq_bufL, right),
            (wq_ref, wq_bufR, left),
            (wo_ref, wo_bufL, right),
            (wo_ref, wo_bufR, left),
        ]):
            r = pltpu.make_async_remote_copy(
                src_ref=src, dst_ref=dst,
                send_sem=p1_ssem.at[idx], recv_sem=p1_rsem.at[idx],
                device_id=(tgt,), device_id_type=pl.DeviceIdType.MESH)
            r.start()
            p1.append(r)

        compute_chunk(0, wq_ref[...], wo_ref[...], first=True)

        for r in p1:
            r.wait_recv()

        p2 = []
        for idx, (src, dst, tgt) in enumerate([
            (wq_bufR.at[pl.ds(0, WQ_HALF)], wq_bufO.at[pl.ds(0, WQ_HALF)], left),
            (wq_bufL.at[pl.ds(WQ_HALF, WQ_HALF)],
             wq_bufO.at[pl.ds(WQ_HALF, WQ_HALF)], right),
            (wo_bufR.at[pl.ds(0, WO_HALF)], wo_bufO.at[pl.ds(0, WO_HALF)], left),
            (wo_bufL.at[pl.ds(WO_HALF, WO_HALF)],
             wo_bufO.at[pl.ds(WO_HALF, WO_HALF)], right),
        ]):
            r = pltpu.make_async_remote_copy(
                src_ref=src, dst_ref=dst,
                send_sem=p2_ssem.at[idx], recv_sem=p2_rsem.at[idx],
                device_id=(tgt,), device_id_type=pl.DeviceIdType.MESH)
            r.start()
            p2.append(r)

        compute_chunk(1, wq_bufL[...], wo_bufL[...], first=False)
        compute_chunk(2, wq_bufR[...], wo_bufR[...], first=False)

        for r in p2:
            r.wait_recv()

        compute_chunk(3, wq_bufO[...], wo_bufO[...], first=False)

        for r in p1 + p2:
            r.wait_send()

        @functools.partial(pl.run_scoped, sem=pltpu.SemaphoreType.REGULAR)
        def _(sem):
            for nbr in (left, right):
                pl.semaphore_signal(sem, inc=1, device_id=(nbr,),
                                    device_id_type=pl.DeviceIdType.MESH)
            pl.semaphore_wait(sem, 2)

    return pl.pallas_call(
        body,
        out_shape=jax.ShapeDtypeStruct((B_PER, SQ, DM), jnp.float32),
        in_specs=[
            pl.BlockSpec(memory_space=pltpu.VMEM),
            pl.BlockSpec(memory_space=pltpu.VMEM),
            pl.BlockSpec(memory_space=pltpu.MemorySpace.HBM),
            pl.BlockSpec(memory_space=pltpu.MemorySpace.HBM),
            pl.BlockSpec(memory_space=pltpu.VMEM),
        ],
        out_specs=pl.BlockSpec(memory_space=pltpu.VMEM),
        scratch_shapes=[
            pltpu.VMEM((DM, F_PER), jnp.bfloat16),
            pltpu.VMEM((DM, F_PER), jnp.bfloat16),
            pltpu.VMEM((DM, F_PER), jnp.bfloat16),
            pltpu.VMEM((F_PER, DM), jnp.bfloat16),
            pltpu.VMEM((F_PER, DM), jnp.bfloat16),
            pltpu.VMEM((F_PER, DM), jnp.bfloat16),
            pltpu.VMEM((N_DEV, B_PER, SKV, F_PER), jnp.bfloat16),
            pltpu.VMEM((N_DEV, B_PER, SKV, F_PER), jnp.bfloat16),
            pltpu.SemaphoreType.DMA((4,)),
            pltpu.SemaphoreType.DMA((4,)),
            pltpu.SemaphoreType.DMA((4,)),
            pltpu.SemaphoreType.DMA((4,)),
            pltpu.SemaphoreType.DMA((4,)),
            pltpu.SemaphoreType.DMA((4,)),
        ],
        compiler_params=pltpu.CompilerParams(
            collective_id=0, vmem_limit_bytes=100 * 1024 * 1024),
    )(x_bf, wq_bf, k_bf, v_bf, wo_bf)
